# Initial kernel scaffold; baseline (speedup 1.0000x reference)
import functools

import jax
import jax.numpy as jnp
from jax import lax
from jax.experimental import pallas as pl
from jax.experimental.pallas import tpu as pltpu
from jax.experimental.pallas import tpu_sc as plsc

N = 10000
E = 160000
D = 256
H = 16

CHUNK = 128
NCHUNKS = E // CHUNK
NCORES = 2
NSUB = 16
NW = NCORES * NSUB
KFULL = NCHUNKS // NW
KTAIL = NCHUNKS - KFULL * NW
ROWS_PT = N // NSUB

_mesh = plsc.VectorSubcoreMesh(core_axis_name="c", subcore_axis_name="s")


def _scatter_body(gather, src_hbm, dst_hbm, g_hbm, zero_hbm, out_hbm,
                  src_v, dst_v, rows_v, acc, sem):
    c = lax.axis_index("c")
    s = lax.axis_index("s")
    wid = c * NSUB + s

    pltpu.sync_copy(zero_hbm.at[pl.ds(s * ROWS_PT, ROWS_PT)],
                    acc.at[pl.ds(s * ROWS_PT, ROWS_PT)])
    plsc.subcore_barrier()

    if not gather:
        pltpu.sync_copy(g_hbm, rows_v)

    def do_chunk(chunk):
        pltpu.sync_copy(src_hbm.at[chunk], src_v)
        pltpu.sync_copy(dst_hbm.at[chunk], dst_v)
        if gather:
            pltpu.async_copy(g_hbm.at[src_v], rows_v, sem).wait()
        pltpu.sync_copy(rows_v, acc.at[dst_v], add=True)

    def body(k, carry):
        do_chunk(k * NW + wid)
        return carry

    lax.fori_loop(0, KFULL, body, 0)

    @pl.when(wid < KTAIL)
    def _():
        do_chunk(KFULL * NW + wid)

    plsc.subcore_barrier()
    pltpu.sync_copy(acc.at[pl.ds(s * ROWS_PT, ROWS_PT)],
                    out_hbm.at[c, pl.ds(s * ROWS_PT, ROWS_PT)])


_sc_scratch = [
    pltpu.VMEM((CHUNK,), jnp.int32),
    pltpu.VMEM((CHUNK,), jnp.int32),
    pltpu.VMEM((CHUNK, H), jnp.float32),
    pltpu.VMEM_SHARED((N, H), jnp.float32),
    pltpu.SemaphoreType.DMA,
]

_prop = pl.kernel(
    functools.partial(_scatter_body, True),
    out_type=jax.ShapeDtypeStruct((NCORES, N, H), jnp.float32),
    mesh=_mesh,
    scratch_types=_sc_scratch,
    name="gcn_edge_prop",
)

_deg = pl.kernel(
    functools.partial(_scatter_body, False),
    out_type=jax.ShapeDtypeStruct((NCORES, N, H), jnp.float32),
    mesh=_mesh,
    scratch_types=_sc_scratch,
    name="gcn_degree",
)


BLK = 2000
_GRID = N // BLK


def _mm1_body(x_ref, w_ref, d0_ref, d1_ref, g_ref, dinv_ref):
    deg = d0_ref[...] + d1_ref[...] + 1.0
    dinv = lax.rsqrt(deg)
    h = jnp.dot(x_ref[...], w_ref[...], preferred_element_type=jnp.float32)
    dinv_ref[...] = dinv
    g_ref[...] = dinv * h


_mm1 = pl.pallas_call(
    _mm1_body,
    grid=(_GRID,),
    in_specs=[
        pl.BlockSpec((BLK, D), lambda i: (i, 0)),
        pl.BlockSpec((D, H), lambda i: (0, 0)),
        pl.BlockSpec((BLK, H), lambda i: (i, 0)),
        pl.BlockSpec((BLK, H), lambda i: (i, 0)),
    ],
    out_specs=[
        pl.BlockSpec((BLK, H), lambda i: (i, 0)),
        pl.BlockSpec((BLK, H), lambda i: (i, 0)),
    ],
    out_shape=[
        jax.ShapeDtypeStruct((N, H), jnp.float32),
        jax.ShapeDtypeStruct((N, H), jnp.float32),
    ],
)


def _mid_body(p0_ref, p1_ref, g_ref, dinv_ref, b_ref, w_ref, gn_ref, h_ref):
    z = dinv_ref[...] * (p0_ref[...] + p1_ref[...] + g_ref[...]) + b_ref[...]
    h = jnp.maximum(z, 0.0)
    hp = jnp.dot(h, w_ref[...], preferred_element_type=jnp.float32)
    h_ref[...] = h
    gn_ref[...] = dinv_ref[...] * hp


_mid = pl.pallas_call(
    _mid_body,
    grid=(_GRID,),
    in_specs=[
        pl.BlockSpec((BLK, H), lambda i: (i, 0)),
        pl.BlockSpec((BLK, H), lambda i: (i, 0)),
        pl.BlockSpec((BLK, H), lambda i: (i, 0)),
        pl.BlockSpec((BLK, H), lambda i: (i, 0)),
        pl.BlockSpec((1, H), lambda i: (0, 0)),
        pl.BlockSpec((H, H), lambda i: (0, 0)),
    ],
    out_specs=[
        pl.BlockSpec((BLK, H), lambda i: (i, 0)),
        pl.BlockSpec((BLK, H), lambda i: (i, 0)),
    ],
    out_shape=[
        jax.ShapeDtypeStruct((N, H), jnp.float32),
        jax.ShapeDtypeStruct((N, H), jnp.float32),
    ],
)


def _fin_body(r0_ref, r1_ref, g_ref, dinv_ref, h2_ref, b3_ref, wa_ref,
              ba_ref, choice_ref, value_ref):
    cfull = dinv_ref[...] * (r0_ref[...] + r1_ref[...] + g_ref[...])
    c = cfull[:, 0:1] + b3_ref[...]
    m = jnp.max(c)
    e = jnp.exp(c - m)
    choice_ref[...] = e / jnp.sum(e)
    v = jnp.mean(h2_ref[...], axis=0, keepdims=True)
    value_ref[...] = jnp.sum(v * wa_ref[...]).reshape(1, 1) + ba_ref[...]


_fin = pl.pallas_call(
    _fin_body,
    in_specs=[pl.BlockSpec(memory_space=pltpu.VMEM)] * 8,
    out_specs=[
        pl.BlockSpec(memory_space=pltpu.VMEM),
        pl.BlockSpec(memory_space=pltpu.VMEM),
    ],
    out_shape=[
        jax.ShapeDtypeStruct((N, 1), jnp.float32),
        jax.ShapeDtypeStruct((1, 1), jnp.float32),
    ],
)


def kernel(x, edge_index, W1, b1, W2, b2, W3, b3, Wa, ba):
    src = edge_index[0].astype(jnp.int32).reshape(NCHUNKS, CHUNK)
    dst = edge_index[1].astype(jnp.int32).reshape(NCHUNKS, CHUNK)
    zeros = jnp.zeros((N, H), jnp.float32)
    ones = jnp.ones((CHUNK, H), jnp.float32)
    w3p = jnp.concatenate([W3, jnp.zeros((H, H - 1), jnp.float32)], axis=1)

    degp = _deg(dst, ones, zeros)
    g1, dinv = _mm1(x, W1, degp[0], degp[1])
    p = _prop(src, dst, g1, zeros)
    g2, _ = _mid(p[0], p[1], g1, dinv, b1.reshape(1, H), W2)
    q = _prop(src, dst, g2, zeros)
    g3, h2 = _mid(q[0], q[1], g2, dinv, b2.reshape(1, H), w3p)
    r = _prop(src, dst, g3, zeros)
    choice, value = _fin(r[0], r[1], g3, dinv, h2,
                         b3.reshape(1, 1), Wa.reshape(1, H),
                         ba.reshape(1, 1))
    return choice.reshape(N), value.reshape(())


# speedup vs baseline: 9.9245x; 9.9245x over previous
import jax
import jax.numpy as jnp
from jax import lax
from jax.experimental import pallas as pl
from jax.experimental.pallas import tpu as pltpu
from jax.experimental.pallas import tpu_sc as plsc

N = 10000
E = 160000
D = 256
H = 16
W = 128

CHUNK = 128
NCHUNKS = E // CHUNK
NCORES = 2
NSUB = 16
NW = NCORES * NSUB
KFULL = NCHUNKS // NW
KTAIL = NCHUNKS - KFULL * NW
NPAD = 10240
ROWS_PT = NPAD // NSUB

_mesh = plsc.VectorSubcoreMesh(core_axis_name="c", subcore_axis_name="s")


def _edge_loop(wid, do_chunk):
    def body(k, carry):
        do_chunk(k * NW + wid)
        return carry

    lax.fori_loop(0, KFULL, body, 0)

    @pl.when(wid < KTAIL)
    def _():
        do_chunk(KFULL * NW + wid)


def _zero_acc(s, zero_hbm, acc):
    off = pl.multiple_of(s * ROWS_PT, ROWS_PT)
    pltpu.sync_copy(zero_hbm.at[pl.ds(off, ROWS_PT)],
                    acc.at[pl.ds(off, ROWS_PT)])
    plsc.subcore_barrier()


def _writeback(c, s, acc, out_hbm):
    plsc.subcore_barrier()
    off = pl.multiple_of(s * ROWS_PT, ROWS_PT)
    pltpu.sync_copy(acc.at[pl.ds(off, ROWS_PT)],
                    out_hbm.at[c, pl.ds(off, ROWS_PT)])


def _prop_body(src_hbm, dst_hbm, g_hbm, zero_hbm, out_hbm,
               src_v, dst_v, rows_v, acc, sem):
    c = lax.axis_index("c")
    s = lax.axis_index("s")
    wid = c * NSUB + s
    _zero_acc(s, zero_hbm, acc)

    def do_chunk(chunk):
        coff = pl.multiple_of(chunk * CHUNK, CHUNK)
        pltpu.sync_copy(src_hbm.at[pl.ds(coff, CHUNK)], src_v)
        pltpu.sync_copy(dst_hbm.at[pl.ds(coff, CHUNK)], dst_v)
        pltpu.async_copy(g_hbm.at[src_v], rows_v, sem).wait()
        pltpu.async_copy(rows_v, acc.at[dst_v], sem, add=True).wait()

    _edge_loop(wid, do_chunk)
    _writeback(c, s, acc, out_hbm)


def _deg_body(dst_hbm, ones_hbm, zero_hbm, out_hbm,
              src_v, dst_v, rows_v, acc, sem):
    c = lax.axis_index("c")
    s = lax.axis_index("s")
    wid = c * NSUB + s
    _zero_acc(s, zero_hbm, acc)
    pltpu.sync_copy(ones_hbm, rows_v)

    def do_chunk(chunk):
        coff = pl.multiple_of(chunk * CHUNK, CHUNK)
        pltpu.sync_copy(dst_hbm.at[pl.ds(coff, CHUNK)], dst_v)
        pltpu.async_copy(rows_v, acc.at[dst_v], sem, add=True).wait()

    _edge_loop(wid, do_chunk)
    _writeback(c, s, acc, out_hbm)


_sc_scratch = [
    pltpu.VMEM((CHUNK,), jnp.int32),
    pltpu.VMEM((CHUNK,), jnp.int32),
    pltpu.VMEM((CHUNK, W), jnp.float32),
    pltpu.VMEM_SHARED((NPAD, W), jnp.float32),
    pltpu.SemaphoreType.DMA,
]

_prop = pl.kernel(
    _prop_body,
    out_type=jax.ShapeDtypeStruct((NCORES, NPAD, W), jnp.float32),
    mesh=_mesh,
    scratch_types=_sc_scratch,
    name="gcn_edge_prop",
)

_deg = pl.kernel(
    _deg_body,
    out_type=jax.ShapeDtypeStruct((NCORES, NPAD, W), jnp.float32),
    mesh=_mesh,
    scratch_types=_sc_scratch,
    name="gcn_degree",
)


BLK = 2000
_GRID = N // BLK


def _mm1_body(x_ref, w_ref, d0_ref, d1_ref, g_ref, dinv_ref):
    deg = d0_ref[...] + d1_ref[...] + 1.0
    dinv = lax.rsqrt(deg)
    h = jnp.dot(x_ref[...], w_ref[...], preferred_element_type=jnp.float32)
    dinv_ref[...] = dinv
    g_ref[...] = jnp.concatenate(
        [dinv * h, jnp.zeros((BLK, W - H), jnp.float32)], axis=1)


_mm1 = pl.pallas_call(
    _mm1_body,
    grid=(_GRID,),
    in_specs=[
        pl.BlockSpec((BLK, D), lambda i: (i, 0)),
        pl.BlockSpec((D, H), lambda i: (0, 0)),
        pl.BlockSpec((BLK, H), lambda i: (i, 0)),
        pl.BlockSpec((BLK, H), lambda i: (i, 0)),
    ],
    out_specs=[
        pl.BlockSpec((BLK, W), lambda i: (i, 0)),
        pl.BlockSpec((BLK, H), lambda i: (i, 0)),
    ],
    out_shape=[
        jax.ShapeDtypeStruct((N, W), jnp.float32),
        jax.ShapeDtypeStruct((N, H), jnp.float32),
    ],
)


def _mid_body(p0_ref, p1_ref, g_ref, dinv_ref, b_ref, w_ref, gn_ref, h_ref):
    tot = p0_ref[...] + p1_ref[...] + g_ref[...]
    z = dinv_ref[...] * tot + b_ref[...]
    h = jnp.maximum(z, 0.0)
    hp = jnp.dot(h, w_ref[...], preferred_element_type=jnp.float32)
    h_ref[...] = h
    gn_ref[...] = jnp.concatenate(
        [dinv_ref[...] * hp, jnp.zeros((BLK, W - H), jnp.float32)], axis=1)


_mid = pl.pallas_call(
    _mid_body,
    grid=(_GRID,),
    in_specs=[
        pl.BlockSpec((BLK, H), lambda i: (i, 0)),
        pl.BlockSpec((BLK, H), lambda i: (i, 0)),
        pl.BlockSpec((BLK, H), lambda i: (i, 0)),
        pl.BlockSpec((BLK, H), lambda i: (i, 0)),
        pl.BlockSpec((1, H), lambda i: (0, 0)),
        pl.BlockSpec((H, H), lambda i: (0, 0)),
    ],
    out_specs=[
        pl.BlockSpec((BLK, W), lambda i: (i, 0)),
        pl.BlockSpec((BLK, H), lambda i: (i, 0)),
    ],
    out_shape=[
        jax.ShapeDtypeStruct((N, W), jnp.float32),
        jax.ShapeDtypeStruct((N, H), jnp.float32),
    ],
)


def _fin_body(r0_ref, r1_ref, g_ref, dinv_ref, h2_ref, b3_ref, wa_ref,
              ba_ref, choice_ref, value_ref):
    cfull = dinv_ref[...] * (r0_ref[...] + r1_ref[...] + g_ref[...])
    c = cfull[:, 0:1] + b3_ref[...]
    m = jnp.max(c)
    e = jnp.exp(c - m)
    choice_ref[...] = e / jnp.sum(e)
    v = jnp.mean(h2_ref[...], axis=0, keepdims=True)
    value_ref[...] = jnp.sum(v * wa_ref[...]).reshape(1, 1) + ba_ref[...]


_fin = pl.pallas_call(
    _fin_body,
    in_specs=[
        pl.BlockSpec((N, H), lambda: (0, 0)),
        pl.BlockSpec((N, H), lambda: (0, 0)),
        pl.BlockSpec((N, H), lambda: (0, 0)),
        pl.BlockSpec((N, H), lambda: (0, 0)),
        pl.BlockSpec((N, H), lambda: (0, 0)),
        pl.BlockSpec((1, 1), lambda: (0, 0)),
        pl.BlockSpec((1, H), lambda: (0, 0)),
        pl.BlockSpec((1, 1), lambda: (0, 0)),
    ],
    out_specs=[
        pl.BlockSpec((N, 1), lambda: (0, 0)),
        pl.BlockSpec((1, 1), lambda: (0, 0)),
    ],
    out_shape=[
        jax.ShapeDtypeStruct((N, 1), jnp.float32),
        jax.ShapeDtypeStruct((1, 1), jnp.float32),
    ],
)


def kernel(x, edge_index, W1, b1, W2, b2, W3, b3, Wa, ba):
    src = edge_index[0].astype(jnp.int32)
    dst = edge_index[1].astype(jnp.int32)
    zeros = jnp.zeros((NPAD, W), jnp.float32)
    ones = jnp.ones((CHUNK, W), jnp.float32)
    w3p = jnp.concatenate([W3, jnp.zeros((H, H - 1), jnp.float32)], axis=1)

    degp = _deg(dst, ones, zeros)
    g1, dinv = _mm1(x, W1, degp[0, :N, :H], degp[1, :N, :H])
    p = _prop(src, dst, g1, zeros)
    g2, _ = _mid(p[0, :N, :H], p[1, :N, :H], g1[:, :H], dinv,
                 b1.reshape(1, H), W2)
    q = _prop(src, dst, g2, zeros)
    g3, h2 = _mid(q[0, :N, :H], q[1, :N, :H], g2[:, :H], dinv,
                  b2.reshape(1, H), w3p)
    r = _prop(src, dst, g3, zeros)
    choice, value = _fin(r[0, :N, :H], r[1, :N, :H], g3[:, :H], dinv, h2,
                         b3.reshape(1, 1), Wa.reshape(1, H),
                         ba.reshape(1, 1))
    return choice.reshape(N), value.reshape(())


# speedup vs baseline: 14.7937x; 1.4906x over previous
import jax
import jax.numpy as jnp
from jax import lax
from jax.experimental import pallas as pl
from jax.experimental.pallas import tpu as pltpu
from jax.experimental.pallas import tpu_sc as plsc

N = 10000
E = 160000
D = 256
H = 16
W = 128

CHUNK = 128
NCHUNKS = E // CHUNK
NCORES = 2
NSUB = 16
NW = NCORES * NSUB
KFULL = NCHUNKS // NW
KTAIL = NCHUNKS - KFULL * NW
PAIRS = (KFULL - 1) // 2
assert KFULL == 2 * PAIRS + 1
NPAD = 10240
ROWS_PT = NPAD // NSUB

_mesh = plsc.VectorSubcoreMesh(core_axis_name="c", subcore_axis_name="s")


def _zero_acc(s, zero_hbm, acc):
    off = pl.multiple_of(s * ROWS_PT, ROWS_PT)
    pltpu.sync_copy(zero_hbm.at[pl.ds(off, ROWS_PT)],
                    acc.at[pl.ds(off, ROWS_PT)])
    plsc.subcore_barrier()


def _writeback(c, s, acc, out_hbm):
    plsc.subcore_barrier()
    off = pl.multiple_of(s * ROWS_PT, ROWS_PT)
    pltpu.sync_copy(acc.at[pl.ds(off, ROWS_PT)],
                    out_hbm.at[c, pl.ds(off, ROWS_PT)])


def _load_idx(wid, src_hbm, idx_v):
    ebase = pl.multiple_of(wid * (KFULL * CHUNK), CHUNK)
    pltpu.sync_copy(src_hbm.at[pl.ds(ebase, KFULL * CHUNK)],
                    idx_v.at[pl.ds(0, KFULL * CHUNK)])

    @pl.when(wid < KTAIL)
    def _():
        toff = pl.multiple_of(NW * KFULL * CHUNK, CHUNK) + wid * CHUNK
        pltpu.sync_copy(src_hbm.at[pl.ds(toff, CHUNK)],
                        idx_v.at[pl.ds(KFULL * CHUNK, CHUNK)])


def _prop_body(src_hbm, dst_hbm, g_hbm, zero_hbm, out_hbm,
               srcs_v, dsts_v, rows_a, rows_b, acc, sga, sgb, ssa, ssb):
    c = lax.axis_index("c")
    s = lax.axis_index("s")
    wid = c * NSUB + s
    _zero_acc(s, zero_hbm, acc)
    _load_idx(wid, src_hbm, srcs_v)
    _load_idx(wid, dst_hbm, dsts_v)

    def src_at(k):
        return srcs_v.at[pl.ds(k * CHUNK, CHUNK)]

    def dst_at(k):
        return dsts_v.at[pl.ds(k * CHUNK, CHUNK)]

    def ig(k, rows, sg):
        pltpu.async_copy(g_hbm.at[src_at(k)], rows, sg)

    def wg(k, rows, sg):
        pltpu.make_async_copy(g_hbm.at[src_at(k)], rows, sg).wait()

    def isc(k, rows, ss):
        pltpu.async_copy(rows, acc.at[dst_at(k)], ss, add=True)

    def wsc(k, rows, ss):
        pltpu.make_async_copy(rows, acc.at[dst_at(k)], ss).wait()

    ig(0, rows_a, sga)

    def pair(p, carry):
        e = 2 * p
        wg(e, rows_a, sga)
        isc(e, rows_a, ssa)

        @pl.when(p > 0)
        def _():
            wsc(e - 1, rows_b, ssb)

        ig(e + 1, rows_b, sgb)
        wg(e + 1, rows_b, sgb)
        isc(e + 1, rows_b, ssb)
        wsc(e, rows_a, ssa)
        ig(e + 2, rows_a, sga)
        return carry

    lax.fori_loop(0, PAIRS, pair, 0)

    last = KFULL - 1
    wg(last, rows_a, sga)
    isc(last, rows_a, ssa)
    wsc(last - 1, rows_b, ssb)

    @pl.when(wid < KTAIL)
    def _():
        ig(KFULL, rows_b, sgb)
        wg(KFULL, rows_b, sgb)
        isc(KFULL, rows_b, ssb)
        wsc(KFULL, rows_b, ssb)

    wsc(last, rows_a, ssa)
    _writeback(c, s, acc, out_hbm)


def _deg_body(dst_hbm, ones_hbm, zero_hbm, out_hbm,
              dsts_v, rows_v, acc, sem):
    c = lax.axis_index("c")
    s = lax.axis_index("s")
    wid = c * NSUB + s
    _zero_acc(s, zero_hbm, acc)
    _load_idx(wid, dst_hbm, dsts_v)
    pltpu.sync_copy(ones_hbm, rows_v)

    def dst_at(k):
        return dsts_v.at[pl.ds(k * CHUNK, CHUNK)]

    def fire(k, carry):
        pltpu.async_copy(rows_v, acc.at[dst_at(k)], sem, add=True)
        return carry

    def drain(k, carry):
        pltpu.make_async_copy(rows_v, acc.at[dst_at(k)], sem).wait()
        return carry

    lax.fori_loop(0, KFULL, fire, 0)

    @pl.when(wid < KTAIL)
    def _():
        fire(KFULL, 0)
        drain(KFULL, 0)

    lax.fori_loop(0, KFULL, drain, 0)
    _writeback(c, s, acc, out_hbm)


_prop = pl.kernel(
    _prop_body,
    out_type=jax.ShapeDtypeStruct((NCORES, NPAD, W), jnp.float32),
    mesh=_mesh,
    scratch_types=[
        pltpu.VMEM(((KFULL + 1) * CHUNK,), jnp.int32),
        pltpu.VMEM(((KFULL + 1) * CHUNK,), jnp.int32),
        pltpu.VMEM((CHUNK, W), jnp.float32),
        pltpu.VMEM((CHUNK, W), jnp.float32),
        pltpu.VMEM_SHARED((NPAD, W), jnp.float32),
        pltpu.SemaphoreType.DMA,
        pltpu.SemaphoreType.DMA,
        pltpu.SemaphoreType.DMA,
        pltpu.SemaphoreType.DMA,
    ],
    name="gcn_edge_prop",
)

_deg = pl.kernel(
    _deg_body,
    out_type=jax.ShapeDtypeStruct((NCORES, NPAD, W), jnp.float32),
    mesh=_mesh,
    scratch_types=[
        pltpu.VMEM(((KFULL + 1) * CHUNK,), jnp.int32),
        pltpu.VMEM((CHUNK, W), jnp.float32),
        pltpu.VMEM_SHARED((NPAD, W), jnp.float32),
        pltpu.SemaphoreType.DMA,
    ],
    name="gcn_degree",
)


BLK = 2000
_GRID = N // BLK


def _mm1_body(x_ref, w_ref, d0_ref, d1_ref, g_ref, dinv_ref):
    deg = d0_ref[...] + d1_ref[...] + 1.0
    dinv = lax.rsqrt(deg)
    h = jnp.dot(x_ref[...], w_ref[...], preferred_element_type=jnp.float32)
    dinv_ref[...] = dinv
    g_ref[...] = jnp.concatenate(
        [dinv * h, jnp.zeros((BLK, W - H), jnp.float32)], axis=1)


_mm1 = pl.pallas_call(
    _mm1_body,
    grid=(_GRID,),
    in_specs=[
        pl.BlockSpec((BLK, D), lambda i: (i, 0)),
        pl.BlockSpec((D, H), lambda i: (0, 0)),
        pl.BlockSpec((BLK, H), lambda i: (i, 0)),
        pl.BlockSpec((BLK, H), lambda i: (i, 0)),
    ],
    out_specs=[
        pl.BlockSpec((BLK, W), lambda i: (i, 0)),
        pl.BlockSpec((BLK, H), lambda i: (i, 0)),
    ],
    out_shape=[
        jax.ShapeDtypeStruct((N, W), jnp.float32),
        jax.ShapeDtypeStruct((N, H), jnp.float32),
    ],
)


def _mid_body(p0_ref, p1_ref, g_ref, dinv_ref, b_ref, w_ref, gn_ref, h_ref):
    tot = p0_ref[...] + p1_ref[...] + g_ref[...]
    z = dinv_ref[...] * tot + b_ref[...]
    h = jnp.maximum(z, 0.0)
    hp = jnp.dot(h, w_ref[...], preferred_element_type=jnp.float32)
    h_ref[...] = h
    gn_ref[...] = jnp.concatenate(
        [dinv_ref[...] * hp, jnp.zeros((BLK, W - H), jnp.float32)], axis=1)


_mid = pl.pallas_call(
    _mid_body,
    grid=(_GRID,),
    in_specs=[
        pl.BlockSpec((BLK, H), lambda i: (i, 0)),
        pl.BlockSpec((BLK, H), lambda i: (i, 0)),
        pl.BlockSpec((BLK, H), lambda i: (i, 0)),
        pl.BlockSpec((BLK, H), lambda i: (i, 0)),
        pl.BlockSpec((1, H), lambda i: (0, 0)),
        pl.BlockSpec((H, H), lambda i: (0, 0)),
    ],
    out_specs=[
        pl.BlockSpec((BLK, W), lambda i: (i, 0)),
        pl.BlockSpec((BLK, H), lambda i: (i, 0)),
    ],
    out_shape=[
        jax.ShapeDtypeStruct((N, W), jnp.float32),
        jax.ShapeDtypeStruct((N, H), jnp.float32),
    ],
)


def _fin_body(r0_ref, r1_ref, g_ref, dinv_ref, h2_ref, b3_ref, wa_ref,
              ba_ref, choice_ref, value_ref):
    cfull = dinv_ref[...] * (r0_ref[...] + r1_ref[...] + g_ref[...])
    c = cfull[:, 0:1] + b3_ref[...]
    m = jnp.max(c)
    e = jnp.exp(c - m)
    choice_ref[...] = e / jnp.sum(e)
    v = jnp.mean(h2_ref[...], axis=0, keepdims=True)
    value_ref[...] = jnp.sum(v * wa_ref[...]).reshape(1, 1) + ba_ref[...]


_fin = pl.pallas_call(
    _fin_body,
    in_specs=[
        pl.BlockSpec((N, H), lambda: (0, 0)),
        pl.BlockSpec((N, H), lambda: (0, 0)),
        pl.BlockSpec((N, H), lambda: (0, 0)),
        pl.BlockSpec((N, H), lambda: (0, 0)),
        pl.BlockSpec((N, H), lambda: (0, 0)),
        pl.BlockSpec((1, 1), lambda: (0, 0)),
        pl.BlockSpec((1, H), lambda: (0, 0)),
        pl.BlockSpec((1, 1), lambda: (0, 0)),
    ],
    out_specs=[
        pl.BlockSpec((N, 1), lambda: (0, 0)),
        pl.BlockSpec((1, 1), lambda: (0, 0)),
    ],
    out_shape=[
        jax.ShapeDtypeStruct((N, 1), jnp.float32),
        jax.ShapeDtypeStruct((1, 1), jnp.float32),
    ],
)


def kernel(x, edge_index, W1, b1, W2, b2, W3, b3, Wa, ba):
    src = edge_index[0].astype(jnp.int32)
    dst = edge_index[1].astype(jnp.int32)
    zeros = jnp.zeros((NPAD, W), jnp.float32)
    ones = jnp.ones((CHUNK, W), jnp.float32)
    w3p = jnp.concatenate([W3, jnp.zeros((H, H - 1), jnp.float32)], axis=1)

    degp = _deg(dst, ones, zeros)
    g1, dinv = _mm1(x, W1, degp[0, :N, :H], degp[1, :N, :H])
    p = _prop(src, dst, g1, zeros)
    g2, _ = _mid(p[0, :N, :H], p[1, :N, :H], g1[:, :H], dinv,
                 b1.reshape(1, H), W2)
    q = _prop(src, dst, g2, zeros)
    g3, h2 = _mid(q[0, :N, :H], q[1, :N, :H], g2[:, :H], dinv,
                  b2.reshape(1, H), w3p)
    r = _prop(src, dst, g3, zeros)
    choice, value = _fin(r[0, :N, :H], r[1, :N, :H], g3[:, :H], dinv, h2,
                         b3.reshape(1, 1), Wa.reshape(1, H),
                         ba.reshape(1, 1))
    return choice.reshape(N), value.reshape(())


# speedup vs baseline: 15.9880x; 1.0807x over previous
import jax
import jax.numpy as jnp
from jax import lax
from jax.experimental import pallas as pl
from jax.experimental.pallas import tpu as pltpu
from jax.experimental.pallas import tpu_sc as plsc

N = 10000
E = 160000
D = 256
H = 16
W = 128

CHUNK = 128
NCHUNKS = E // CHUNK
NCORES = 2
NSUB = 16
NW = NCORES * NSUB
KFULL = NCHUNKS // NW
KTAIL = NCHUNKS - KFULL * NW
PAIRS = (KFULL - 1) // 2
assert KFULL == 2 * PAIRS + 1
NPAD = 10240
ROWS_PT = NPAD // NSUB

_mesh = plsc.VectorSubcoreMesh(core_axis_name="c", subcore_axis_name="s")


def _zero_acc(s, zero_hbm, acc):
    off = pl.multiple_of(s * ROWS_PT, ROWS_PT)
    pltpu.sync_copy(zero_hbm.at[pl.ds(off, ROWS_PT)],
                    acc.at[pl.ds(off, ROWS_PT)])
    plsc.subcore_barrier()


def _writeback(c, s, acc, out_hbm):
    plsc.subcore_barrier()
    off = pl.multiple_of(s * ROWS_PT, ROWS_PT)
    pltpu.sync_copy(acc.at[pl.ds(off, ROWS_PT)],
                    out_hbm.at[c, pl.ds(off, ROWS_PT)])


def _load_idx(wid, src_hbm, idx_v):
    ebase = pl.multiple_of(wid * (KFULL * CHUNK), CHUNK)
    pltpu.sync_copy(src_hbm.at[pl.ds(ebase, KFULL * CHUNK)],
                    idx_v.at[pl.ds(0, KFULL * CHUNK)])

    @pl.when(wid < KTAIL)
    def _():
        toff = pl.multiple_of(NW * KFULL * CHUNK, CHUNK) + wid * CHUNK
        pltpu.sync_copy(src_hbm.at[pl.ds(toff, CHUNK)],
                        idx_v.at[pl.ds(KFULL * CHUNK, CHUNK)])


def _prop_body(src_hbm, dst_hbm, g_hbm, zero_hbm, out_hbm,
               srcs_v, dsts_v, rows_a, rows_b, acc, sga, sgb, ssa, ssb):
    c = lax.axis_index("c")
    s = lax.axis_index("s")
    wid = c * NSUB + s
    _zero_acc(s, zero_hbm, acc)
    _load_idx(wid, src_hbm, srcs_v)
    _load_idx(wid, dst_hbm, dsts_v)

    def src_at(k):
        return srcs_v.at[pl.ds(k * CHUNK, CHUNK)]

    def dst_at(k):
        return dsts_v.at[pl.ds(k * CHUNK, CHUNK)]

    def ig(k, rows, sg):
        pltpu.async_copy(g_hbm.at[src_at(k)], rows, sg)

    def wg(k, rows, sg):
        pltpu.make_async_copy(g_hbm.at[src_at(k)], rows, sg).wait()

    def isc(k, rows, ss):
        pltpu.async_copy(rows, acc.at[dst_at(k)], ss, add=True)

    def wsc(k, rows, ss):
        pltpu.make_async_copy(rows, acc.at[dst_at(k)], ss).wait()

    ig(0, rows_a, sga)

    def pair(p, carry):
        e = 2 * p
        wg(e, rows_a, sga)
        isc(e, rows_a, ssa)

        @pl.when(p > 0)
        def _():
            wsc(e - 1, rows_b, ssb)

        ig(e + 1, rows_b, sgb)
        wg(e + 1, rows_b, sgb)
        isc(e + 1, rows_b, ssb)
        wsc(e, rows_a, ssa)
        ig(e + 2, rows_a, sga)
        return carry

    lax.fori_loop(0, PAIRS, pair, 0)

    last = KFULL - 1
    wg(last, rows_a, sga)
    isc(last, rows_a, ssa)
    wsc(last - 1, rows_b, ssb)

    @pl.when(wid < KTAIL)
    def _():
        ig(KFULL, rows_b, sgb)
        wg(KFULL, rows_b, sgb)
        isc(KFULL, rows_b, ssb)
        wsc(KFULL, rows_b, ssb)

    wsc(last, rows_a, ssa)
    _writeback(c, s, acc, out_hbm)


def _deg_body(dst_hbm, ones_hbm, zero_hbm, out_hbm,
              dsts_v, rows_v, acc, sem):
    c = lax.axis_index("c")
    s = lax.axis_index("s")
    wid = c * NSUB + s
    _zero_acc(s, zero_hbm, acc)
    _load_idx(wid, dst_hbm, dsts_v)
    pltpu.sync_copy(ones_hbm, rows_v)

    def dst_at(k):
        return dsts_v.at[pl.ds(k * CHUNK, CHUNK)]

    def fire(k, carry):
        pltpu.async_copy(rows_v, acc.at[dst_at(k)], sem, add=True)
        return carry

    def drain(k, carry):
        pltpu.make_async_copy(rows_v, acc.at[dst_at(k)], sem).wait()
        return carry

    lax.fori_loop(0, KFULL, fire, 0)

    @pl.when(wid < KTAIL)
    def _():
        fire(KFULL, 0)
        drain(KFULL, 0)

    lax.fori_loop(0, KFULL, drain, 0)
    _writeback(c, s, acc, out_hbm)


_prop = pl.kernel(
    _prop_body,
    out_type=jax.ShapeDtypeStruct((NCORES, NPAD, W), jnp.float32),
    mesh=_mesh,
    scratch_types=[
        pltpu.VMEM(((KFULL + 1) * CHUNK,), jnp.int32),
        pltpu.VMEM(((KFULL + 1) * CHUNK,), jnp.int32),
        pltpu.VMEM((CHUNK, W), jnp.float32),
        pltpu.VMEM((CHUNK, W), jnp.float32),
        pltpu.VMEM_SHARED((NPAD, W), jnp.float32),
        pltpu.SemaphoreType.DMA,
        pltpu.SemaphoreType.DMA,
        pltpu.SemaphoreType.DMA,
        pltpu.SemaphoreType.DMA,
    ],
    name="gcn_edge_prop",
)

_deg = pl.kernel(
    _deg_body,
    out_type=jax.ShapeDtypeStruct((NCORES, NPAD, W), jnp.float32),
    mesh=_mesh,
    scratch_types=[
        pltpu.VMEM(((KFULL + 1) * CHUNK,), jnp.int32),
        pltpu.VMEM((CHUNK, W), jnp.float32),
        pltpu.VMEM_SHARED((NPAD, W), jnp.float32),
        pltpu.SemaphoreType.DMA,
    ],
    name="gcn_degree",
)


BLK = 2000
_GRID = N // BLK


def _mm1_body(x_ref, w_ref, deg_ref, g_ref, dinv_ref):
    deg = deg_ref[0, :, :H] + deg_ref[1, :, :H] + 1.0
    dinv = lax.rsqrt(deg)
    h = jnp.dot(x_ref[...], w_ref[...], preferred_element_type=jnp.float32)
    dinv_ref[...] = dinv
    g_ref[...] = jnp.concatenate(
        [dinv * h, jnp.zeros((BLK, W - H), jnp.float32)], axis=1)


_mm1 = pl.pallas_call(
    _mm1_body,
    grid=(_GRID,),
    in_specs=[
        pl.BlockSpec((BLK, D), lambda i: (i, 0)),
        pl.BlockSpec((D, H), lambda i: (0, 0)),
        pl.BlockSpec((NCORES, BLK, W), lambda i: (0, i, 0)),
    ],
    out_specs=[
        pl.BlockSpec((BLK, W), lambda i: (i, 0)),
        pl.BlockSpec((BLK, H), lambda i: (i, 0)),
    ],
    out_shape=[
        jax.ShapeDtypeStruct((N, W), jnp.float32),
        jax.ShapeDtypeStruct((N, H), jnp.float32),
    ],
)


def _mid_body(p_ref, g_ref, dinv_ref, b_ref, w_ref, gn_ref, h_ref):
    tot = p_ref[0, :, :H] + p_ref[1, :, :H] + g_ref[:, :H]
    z = dinv_ref[...] * tot + b_ref[...]
    h = jnp.maximum(z, 0.0)
    hp = jnp.dot(h, w_ref[...], preferred_element_type=jnp.float32)
    h_ref[...] = h
    gn_ref[...] = jnp.concatenate(
        [dinv_ref[...] * hp, jnp.zeros((BLK, W - H), jnp.float32)], axis=1)


_mid = pl.pallas_call(
    _mid_body,
    grid=(_GRID,),
    in_specs=[
        pl.BlockSpec((NCORES, BLK, W), lambda i: (0, i, 0)),
        pl.BlockSpec((BLK, W), lambda i: (i, 0)),
        pl.BlockSpec((BLK, H), lambda i: (i, 0)),
        pl.BlockSpec((1, H), lambda i: (0, 0)),
        pl.BlockSpec((H, H), lambda i: (0, 0)),
    ],
    out_specs=[
        pl.BlockSpec((BLK, W), lambda i: (i, 0)),
        pl.BlockSpec((BLK, H), lambda i: (i, 0)),
    ],
    out_shape=[
        jax.ShapeDtypeStruct((N, W), jnp.float32),
        jax.ShapeDtypeStruct((N, H), jnp.float32),
    ],
)


def _fin_body(r_ref, g_ref, dinv_ref, h2_ref, b3_ref, wa_ref,
              ba_ref, choice_ref, value_ref):
    cfull = dinv_ref[...] * (r_ref[0, :, :H] + r_ref[1, :, :H]
                             + g_ref[:, :H])
    c = cfull[:, 0:1] + b3_ref[...]
    m = jnp.max(c)
    e = jnp.exp(c - m)
    choice_ref[...] = e / jnp.sum(e)
    v = jnp.mean(h2_ref[...], axis=0, keepdims=True)
    value_ref[...] = jnp.sum(v * wa_ref[...]).reshape(1, 1) + ba_ref[...]


_fin = pl.pallas_call(
    _fin_body,
    grid=(1,),
    in_specs=[
        pl.BlockSpec((NCORES, N, W), lambda i: (0, 0, 0)),
        pl.BlockSpec((N, W), lambda i: (0, 0)),
        pl.BlockSpec((N, H), lambda i: (0, 0)),
        pl.BlockSpec((N, H), lambda i: (0, 0)),
        pl.BlockSpec((1, 1), lambda i: (0, 0)),
        pl.BlockSpec((1, H), lambda i: (0, 0)),
        pl.BlockSpec((1, 1), lambda i: (0, 0)),
    ],
    out_specs=[
        pl.BlockSpec((N, 1), lambda i: (0, 0)),
        pl.BlockSpec((1, 1), lambda i: (0, 0)),
    ],
    out_shape=[
        jax.ShapeDtypeStruct((N, 1), jnp.float32),
        jax.ShapeDtypeStruct((1, 1), jnp.float32),
    ],
)


def kernel(x, edge_index, W1, b1, W2, b2, W3, b3, Wa, ba):
    src = edge_index[0].astype(jnp.int32)
    dst = edge_index[1].astype(jnp.int32)
    zeros = jnp.zeros((NPAD, W), jnp.float32)
    ones = jnp.ones((CHUNK, W), jnp.float32)
    w3p = jnp.concatenate([W3, jnp.zeros((H, H - 1), jnp.float32)], axis=1)

    degp = _deg(dst, ones, zeros)
    g1, dinv = _mm1(x, W1, degp)
    p = _prop(src, dst, g1, zeros)
    g2, _ = _mid(p, g1, dinv, b1.reshape(1, H), W2)
    q = _prop(src, dst, g2, zeros)
    g3, h2 = _mid(q, g2, dinv, b2.reshape(1, H), w3p)
    r = _prop(src, dst, g3, zeros)
    choice, value = _fin(r, g3, dinv, h2,
                         b3.reshape(1, 1), Wa.reshape(1, H),
                         ba.reshape(1, 1))
    return choice.reshape(N), value.reshape(())
